# baseline (speedup 1.0000x reference)
import functools

import jax
import jax.numpy as jnp
from jax import lax
from jax.experimental import pallas as pl
from jax.experimental.pallas import tpu as pltpu
from jax.experimental.pallas import tpu_sc as plsc

_D = 64
_V = 1000000
_NC = 2
_NS = 16
_NW = _NC * _NS
_CHUNK = 128
_NBUF = 4
_LEAD = 2

_NBLK_FULL = _V // 128
_TAIL = _V - _NBLK_FULL * 128
_TPW = (_NBLK_FULL + _NW - 1) // _NW


@functools.lru_cache(maxsize=None)
def _make_transpose():
    mesh = plsc.VectorSubcoreMesh(core_axis_name="c", subcore_axis_name="s")

    @functools.partial(
        pl.kernel,
        mesh=mesh,
        out_type=jax.ShapeDtypeStruct((_V * _D // 128, 128), jnp.float32),
        scratch_types=[
            pltpu.VMEM((2, 8, 8, 128), jnp.float32),
            pltpu.VMEM((2, 8, 8, 128), jnp.float32),
            pltpu.SemaphoreType.DMA((2,)),
            pltpu.SemaphoreType.DMA((2,)),
        ],
        compiler_params=pltpu.CompilerParams(
            use_tc_tiling_on_sc=True, needs_layout_passes=False
        ),
    )
    def kt(tt_hbm, tail_hbm, tlin_hbm, in3, out3, isem, osem):
        w = lax.axis_index("s") * _NC + lax.axis_index("c")
        iota = jnp.arange(16, dtype=jnp.int32)
        ish3 = iota >> 3
        i1 = iota & 7
        i0q = [ish3 + 2 * q for q in range(4)]

        def blk_of(j):
            return (j * _NW + w) % _NBLK_FULL

        def fire_in(blk, b):
            for r in range(8):
                pltpu.async_copy(
                    tt_hbm.at[pl.ds(8 * r, 8), pl.ds(blk * 128, 128)],
                    in3.at[b, r], isem.at[b],
                )

        def wait_in(b):
            for r in range(8):
                pltpu.make_async_copy(
                    tt_hbm.at[pl.ds(0, 8), pl.ds(0, 128)], in3.at[b, r], isem.at[b]
                ).wait()

        def fire_out(blk, b):
            for t in range(8):
                pltpu.async_copy(
                    out3.at[b, t], tlin_hbm.at[pl.ds(blk * 64 + 8 * t, 8)], osem.at[b]
                )

        def wait_out(b):
            for t in range(8):
                pltpu.make_async_copy(
                    out3.at[b, t], tlin_hbm.at[pl.ds(0, 8)], osem.at[b]
                ).wait()

        def transpose_block(b, nvocab):
            def vbody(v, _):
                i2 = jnp.full((16,), v, jnp.int32)
                vh = v >> 1
                dt = vh >> 3
                dr = vh & 7
                col0 = (v & 1) * 64
                for q in range(4):
                    vals = plsc.load_gather(in3.at[b], [i0q[q], i1, i2])
                    out3[b, dt, dr, pl.ds(col0 + 16 * q, 16)] = vals
                return 0

            lax.fori_loop(0, nvocab, vbody, 0)

        fire_in(blk_of(0), 0)

        def body(j, _):
            b = j & 1
            fire_in(blk_of(j + 1), 1 - b)
            wait_in(b)

            @pl.when(j >= 2)
            def _():
                wait_out(b)

            transpose_block(b, 128)
            fire_out(blk_of(j), b)
            return 0

        lax.fori_loop(0, _TPW, body, 0, unroll=False)
        wait_out(0)
        wait_out(1)
        wait_in(_TPW & 1)

        @pl.when(w == 0)
        def _():
            for t in range(4):
                pltpu.async_copy(
                    tail_hbm.at[pl.ds(8 * t, 8)], in3.at[0, t], isem.at[0]
                )
            for t in range(4):
                pltpu.make_async_copy(
                    tail_hbm.at[pl.ds(0, 8)], in3.at[0, t], isem.at[0]
                ).wait()
            for t in range(4):
                pltpu.async_copy(
                    in3.at[0, t],
                    tlin_hbm.at[pl.ds(_NBLK_FULL * 64 + 8 * t, 8)], osem.at[0],
                )
            for t in range(4):
                pltpu.make_async_copy(
                    in3.at[0, t], tlin_hbm.at[pl.ds(0, 8)], osem.at[0]
                ).wait()

    return kt


@functools.lru_cache(maxsize=None)
def _make_gather(n_total: int):
    per_w = n_total // _NW
    n_chunk = per_w // _CHUNK
    assert n_chunk % _NBUF == 0 and n_chunk >= 2 * _NBUF
    mesh = plsc.VectorSubcoreMesh(core_axis_name="c", subcore_axis_name="s")

    @functools.partial(
        pl.kernel,
        mesh=mesh,
        out_type=jax.ShapeDtypeStruct((n_total, _D), jnp.float32),
        scratch_types=[
            pltpu.VMEM((n_chunk, _CHUNK), jnp.int32),
            pltpu.VMEM((_NBUF, _CHUNK, _D), jnp.float32),
            pltpu.SemaphoreType.DMA((_NBUF,)),
            pltpu.SemaphoreType.DMA((_NBUF,)),
        ],
        compiler_params=pltpu.CompilerParams(use_tc_tiling_on_sc=False),
    )
    def k(idx_hbm, table_hbm, out_hbm, idx_v, rows_v, gsem, ssem):
        wid = lax.axis_index("s") * _NC + lax.axis_index("c")
        base = wid * per_w
        pltpu.sync_copy(idx_hbm.at[wid], idx_v)

        def fire_gather(j, b):
            pltpu.async_copy(table_hbm.at[idx_v.at[j]], rows_v.at[b], gsem.at[b])

        def wait_gather(b):
            pltpu.make_async_copy(
                table_hbm.at[pl.ds(0, _CHUNK)], rows_v.at[b], gsem.at[b]
            ).wait()

        def fire_store(j, b):
            pltpu.async_copy(
                rows_v.at[b], out_hbm.at[pl.ds(base + j * _CHUNK, _CHUNK)], ssem.at[b]
            )

        def wait_store(b):
            pltpu.make_async_copy(
                rows_v.at[b], out_hbm.at[pl.ds(base, _CHUNK)], ssem.at[b]
            ).wait()

        for j in range(_LEAD):
            fire_gather(j, j % _NBUF)
        for j in range(_NBUF):
            b = j % _NBUF
            b2 = (j + _LEAD) % _NBUF
            if j + _LEAD >= _NBUF:
                wait_store(b2)
            fire_gather(j + _LEAD, b2)
            wait_gather(b)
            fire_store(j, b)

        def body(outer, _):
            for b in range(_NBUF):
                j = outer * _NBUF + b
                b2 = (b + _LEAD) % _NBUF
                wait_store(b2)
                fire_gather(j + _LEAD, b2)
                wait_gather(b)
                fire_store(j, b)
            return 0

        lax.fori_loop(1, n_chunk // _NBUF - 1, body, 0, unroll=False)

        for j in range(n_chunk - _NBUF, n_chunk):
            b = j % _NBUF
            b2 = (j + _LEAD) % _NBUF
            if j + _LEAD < n_chunk:
                wait_store(b2)
                fire_gather(j + _LEAD, b2)
            wait_gather(b)
            fire_store(j, b)
        for b in range(_NBUF):
            wait_store(b)

    return k


def kernel(inputs, embedding):
    b, h = inputs.shape
    n = b * h
    tail = embedding[_NBLK_FULL * 128:].reshape(_TAIL // 2, 128)
    tlin = _make_transpose()(embedding.T, tail)
    table = tlin.reshape(_V, _D)
    idx = inputs.reshape(_NW, n // _NW // _CHUNK, _CHUNK).astype(jnp.int32)
    out = _make_gather(n)(idx, table)
    return out.reshape(b, h, _D)


# speedup vs baseline: 1.7201x; 1.7201x over previous
import functools

import jax
import jax.numpy as jnp
from jax import lax
from jax.experimental import pallas as pl
from jax.experimental.pallas import tpu as pltpu
from jax.experimental.pallas import tpu_sc as plsc

_D = 64
_NC = 2
_NS = 16
_NW = _NC * _NS
_CHUNK = 256
_NBUF = 4
_LEAD = 2


@functools.lru_cache(maxsize=None)
def _make_gather(n_total: int):
    per_w = n_total // _NW
    n_chunk = per_w // _CHUNK
    assert n_chunk % _NBUF == 0 and n_chunk >= 2 * _NBUF
    mesh = plsc.VectorSubcoreMesh(core_axis_name="c", subcore_axis_name="s")

    @functools.partial(
        pl.kernel,
        mesh=mesh,
        out_type=jax.ShapeDtypeStruct((n_total, _D), jnp.float32),
        scratch_types=[
            pltpu.VMEM((n_chunk, _CHUNK), jnp.int32),
            pltpu.VMEM((_NBUF, _CHUNK, _D), jnp.float32),
            pltpu.SemaphoreType.DMA((_NBUF,)),
            pltpu.SemaphoreType.DMA((_NBUF,)),
        ],
        compiler_params=pltpu.CompilerParams(use_tc_tiling_on_sc=False),
    )
    def k(idx_hbm, table_hbm, out_hbm, idx_v, rows_v, gsem, ssem):
        wid = lax.axis_index("s") * _NC + lax.axis_index("c")
        base = wid * per_w
        pltpu.sync_copy(idx_hbm.at[wid], idx_v)

        def fire_gather(j, b):
            pltpu.async_copy(table_hbm.at[idx_v.at[j]], rows_v.at[b], gsem.at[b])

        def wait_gather(b):
            pltpu.make_async_copy(
                table_hbm.at[pl.ds(0, _CHUNK)], rows_v.at[b], gsem.at[b]
            ).wait()

        def fire_store(j, b):
            pltpu.async_copy(
                rows_v.at[b], out_hbm.at[pl.ds(base + j * _CHUNK, _CHUNK)], ssem.at[b]
            )

        def wait_store(b):
            pltpu.make_async_copy(
                rows_v.at[b], out_hbm.at[pl.ds(base, _CHUNK)], ssem.at[b]
            ).wait()

        for j in range(_LEAD):
            fire_gather(j, j % _NBUF)
        for j in range(_NBUF):
            b = j % _NBUF
            b2 = (j + _LEAD) % _NBUF
            if j + _LEAD >= _NBUF:
                wait_store(b2)
            fire_gather(j + _LEAD, b2)
            wait_gather(b)
            fire_store(j, b)

        def body(outer, _):
            for b in range(_NBUF):
                j = outer * _NBUF + b
                b2 = (b + _LEAD) % _NBUF
                wait_store(b2)
                fire_gather(j + _LEAD, b2)
                wait_gather(b)
                fire_store(j, b)
            return 0

        lax.fori_loop(1, n_chunk // _NBUF - 1, body, 0, unroll=False)

        for j in range(n_chunk - _NBUF, n_chunk):
            b = j % _NBUF
            b2 = (j + _LEAD) % _NBUF
            if j + _LEAD < n_chunk:
                wait_store(b2)
                fire_gather(j + _LEAD, b2)
            wait_gather(b)
            fire_store(j, b)
        for b in range(_NBUF):
            wait_store(b)

    return k


def kernel(inputs, embedding):
    b, h = inputs.shape
    n = b * h
    idx = inputs.reshape(_NW, n // _NW // _CHUNK, _CHUNK).astype(jnp.int32)
    out = _make_gather(n)(idx, embedding)
    return out.reshape(b, h, _D)
